# baseline (speedup 1.0000x reference)
import functools

import jax
import jax.numpy as jnp
from jax import lax
from jax.experimental import pallas as pl
from jax.experimental.pallas import tpu as pltpu
from jax.experimental.pallas import tpu_sc as plsc

B = 4096
S = 200
D_MODEL = 64
SCALE = float(D_MODEL) ** 0.5
NC = 2
NS = 16
NW = NC * NS
L = 16

BBLK = 128
NBUF = 2
UNITS = S * (B // BBLK)
UNITS_PER_W = UNITS // NW


@functools.cache
def _build(vocab: int):
    mesh = plsc.VectorSubcoreMesh(core_axis_name="c", subcore_axis_name="s")

    @functools.partial(
        pl.kernel,
        mesh=mesh,
        out_type=jax.ShapeDtypeStruct((S, 8, B // BBLK, 8, BBLK), jnp.float32),
        scratch_types=[
            pltpu.VMEM((NBUF, BBLK), jnp.int32),
            pltpu.VMEM((NBUF, BBLK, D_MODEL), jnp.float32),
            pltpu.VMEM((NBUF, 8, 8, BBLK), jnp.float32),
            pltpu.SemaphoreType.DMA,
            pltpu.SemaphoreType.DMA,
            pltpu.SemaphoreType.DMA,
            pltpu.SemaphoreType.DMA,
        ],
        compiler_params=pltpu.CompilerParams(use_tc_tiling_on_sc=False,
                                             needs_layout_passes=False),
    )
    def emb(tokens_hbm, table_hbm, out_hbm, tv, rows_v, blk_v,
            gsem0, gsem1, ssem0, ssem1):
        gsems = (gsem0, gsem1)
        ssems = (ssem0, ssem1)
        wid = lax.axis_index("s") * NC + lax.axis_index("c")
        u0 = wid * UNITS_PER_W
        iota = lax.iota(jnp.int32, L)

        def unit_su(u):
            s = u // (B // BBLK)
            bt = u % (B // BBLK)
            return s, bt

        def fire_gather(u, p):
            s, bt = unit_su(u)
            pltpu.sync_copy(tokens_hbm.at[s, pl.ds(bt * BBLK, BBLK)],
                            tv.at[p])
            pltpu.async_copy(table_hbm.at[tv.at[p]], rows_v.at[p], gsems[p])

        def wait_gather(p):
            pltpu.make_async_copy(table_hbm.at[tv.at[p]], rows_v.at[p],
                                  gsems[p]).wait()

        def fire_store(u, p):
            s, bt = unit_su(u)
            pltpu.async_copy(blk_v.at[p], out_hbm.at[s, :, bt, :, :],
                             ssems[p])

        def wait_store(p):
            pltpu.make_async_copy(blk_v.at[p], out_hbm.at[0, :, 0, :, :],
                                  ssems[p]).wait()

        def transpose_scale(p):
            def dt_body(dt, acc):
                for ds in range(8):
                    d = dt * 8 + ds
                    dvec = jnp.full((L,), d, jnp.int32)
                    for c in range(BBLK // L):
                        vals = plsc.load_gather(
                            rows_v.at[p], [iota + (c * L), dvec])
                        blk_v[p, dt, ds, pl.ds(c * L, L)] = vals * SCALE
                return acc

            lax.fori_loop(0, 8, dt_body, 0, unroll=False)

        for p in range(NBUF):
            fire_gather(u0 + p, p)

        for p in range(NBUF):
            wait_gather(p)
            transpose_scale(p)
            fire_store(u0 + p, p)
            fire_gather(u0 + NBUF + p, p)

        def group_body(gi, acc):
            for p in range(NBUF):
                k = gi * NBUF + p
                wait_gather(p)
                wait_store(p)
                transpose_scale(p)
                fire_store(u0 + k, p)
                fire_gather(u0 + k + NBUF, p)
            return acc

        lax.fori_loop(1, UNITS_PER_W // NBUF - 1, group_body, 0,
                      unroll=False)

        for p in range(NBUF):
            k = UNITS_PER_W - NBUF + p
            wait_gather(p)
            wait_store(p)
            transpose_scale(p)
            fire_store(u0 + k, p)
        for p in range(NBUF):
            wait_store(p)

    return emb


def kernel(tokens, table):
    vocab, d = table.shape
    tokens_t = tokens.T.astype(jnp.int32)
    out5 = _build(vocab)(tokens_t, table)
    out = out5.transpose(2, 4, 0, 1, 3).reshape(B, S, D_MODEL)
    return out


# speedup vs baseline: 1.5346x; 1.5346x over previous
import functools

import jax
import jax.numpy as jnp
from jax import lax
from jax.experimental import pallas as pl
from jax.experimental.pallas import tpu as pltpu
from jax.experimental.pallas import tpu_sc as plsc

B = 4096
S = 200
D_MODEL = 64
SCALE = float(D_MODEL) ** 0.5
NC = 2
NS = 16
NW = NC * NS
L = 16

BBLK = 128
NBUF = 2
UNITS = S * (B // BBLK)
UNITS_PER_W = UNITS // NW


@functools.cache
def _build(vocab: int):
    mesh = plsc.VectorSubcoreMesh(core_axis_name="c", subcore_axis_name="s")

    @functools.partial(
        pl.kernel,
        mesh=mesh,
        out_type=jax.ShapeDtypeStruct((S, 8, B // BBLK, 8, BBLK), jnp.float32),
        scratch_types=[
            pltpu.VMEM((NBUF, BBLK), jnp.int32),
            pltpu.VMEM((NBUF, BBLK, D_MODEL), jnp.float32),
            pltpu.VMEM((NBUF, 8, 8, BBLK), jnp.float32),
            pltpu.SemaphoreType.DMA,
            pltpu.SemaphoreType.DMA,
            pltpu.SemaphoreType.DMA,
            pltpu.SemaphoreType.DMA,
        ],
        compiler_params=pltpu.CompilerParams(use_tc_tiling_on_sc=False,
                                             needs_layout_passes=False),
    )
    def emb(tokens_hbm, table_hbm, out_hbm, tv, rows_v, blk_v,
            gsem0, gsem1, ssem0, ssem1):
        gsems = (gsem0, gsem1)
        ssems = (ssem0, ssem1)
        wid = lax.axis_index("s") * NC + lax.axis_index("c")
        u0 = wid * UNITS_PER_W
        iota = lax.iota(jnp.int32, L)

        def unit_su(u):
            s = u // (B // BBLK)
            bt = u % (B // BBLK)
            return s, bt

        def fire_gather(u, p):
            s, bt = unit_su(u)
            pltpu.sync_copy(tokens_hbm.at[s, pl.ds(bt * BBLK, BBLK)],
                            tv.at[p])
            pltpu.async_copy(table_hbm.at[tv.at[p]], rows_v.at[p], gsems[p])

        def wait_gather(p):
            pltpu.make_async_copy(table_hbm.at[tv.at[p]], rows_v.at[p],
                                  gsems[p]).wait()

        def fire_store(u, p):
            s, bt = unit_su(u)
            pltpu.async_copy(blk_v.at[p], out_hbm.at[s, :, bt, :, :],
                             ssems[p])

        def wait_store(p):
            pltpu.make_async_copy(blk_v.at[p], out_hbm.at[0, :, 0, :, :],
                                  ssems[p]).wait()

        def transpose_scale(p):
            @plsc.parallel_loop(0, D_MODEL, 1, unroll=4)
            def d_body(d):
                dt = d // 8
                ds = d % 8
                dvec = jnp.full((L,), d, jnp.int32)
                for c in range(BBLK // L):
                    vals = plsc.load_gather(
                        rows_v.at[p], [iota + (c * L), dvec])
                    blk_v[p, dt, ds, pl.ds(c * L, L)] = vals * SCALE

        for p in range(NBUF):
            fire_gather(u0 + p, p)

        for p in range(NBUF):
            wait_gather(p)
            transpose_scale(p)
            fire_store(u0 + p, p)
            fire_gather(u0 + NBUF + p, p)

        def group_body(gi, acc):
            for p in range(NBUF):
                k = gi * NBUF + p
                wait_gather(p)
                wait_store(p)
                transpose_scale(p)
                fire_store(u0 + k, p)
                fire_gather(u0 + k + NBUF, p)
            return acc

        lax.fori_loop(1, UNITS_PER_W // NBUF - 1, group_body, 0,
                      unroll=False)

        for p in range(NBUF):
            k = UNITS_PER_W - NBUF + p
            wait_gather(p)
            wait_store(p)
            transpose_scale(p)
            fire_store(u0 + k, p)
        for p in range(NBUF):
            wait_store(p)

    return emb


def kernel(tokens, table):
    vocab, d = table.shape
    tokens_t = tokens.T.astype(jnp.int32)
    out5 = _build(vocab)(tokens_t, table)
    out = out5.transpose(2, 4, 0, 1, 3).reshape(B, S, D_MODEL)
    return out
